# Initial kernel scaffold; baseline (speedup 1.0000x reference)
import jax
import jax.numpy as jnp
from jax.experimental import pallas as pl


def kernel(x, edge_index, edge_weight, W1_0, b1_0, W2_0, b2_0, W1_1, b1_1, W2_1, b2_1, W1_2, b1_2, W2_2, b2_2):
    raise NotImplementedError("write your pallas kernel here")


# speedup vs baseline: 4.5212x; 4.5212x over previous
import functools

import jax
import jax.numpy as jnp
from jax import lax
from jax.experimental import pallas as pl
from jax.experimental.pallas import tpu as pltpu
from jax.experimental.pallas import tpu_sc as plsc

N = 10000
Np = 10240
E = 160000
D = 256
H = 128
NC = 2
NS = 16
EPT = E // NS
CH = 80
NCH = EPT // CH
RPT = Np // NS
LAYERS = 3


def _sc_aggregate(table2n, srcp, dstp):
    mesh = plsc.VectorSubcoreMesh(core_axis_name="c", subcore_axis_name="s")

    @functools.partial(
        pl.kernel,
        out_type=jax.ShapeDtypeStruct((2 * Np, H), jnp.float32),
        mesh=mesh,
        scratch_types=[
            pltpu.VMEM_SHARED((Np, H), jnp.float32),
            pltpu.VMEM((NCH, CH), jnp.int32),
            pltpu.VMEM((NCH, CH), jnp.int32),
            pltpu.VMEM((CH, H), jnp.float32),
            pltpu.SemaphoreType.DMA,
        ],
    )
    def agg_kernel(table_hbm, srcp_hbm, dstp_hbm, out_hbm,
                   acc, idxs, idxd, rows, sem):
        c = lax.axis_index("c")
        w = lax.axis_index("s")
        pltpu.sync_copy(table_hbm.at[pl.ds(c * Np + w * RPT, RPT)],
                        acc.at[pl.ds(w * RPT, RPT)])
        pltpu.sync_copy(srcp_hbm.at[c, w], idxs)
        pltpu.sync_copy(dstp_hbm.at[w], idxd)
        plsc.subcore_barrier()

        def chunk(j, carry):
            pltpu.async_copy(table_hbm.at[idxs.at[j]], rows, sem).wait()
            pltpu.sync_copy(rows, acc.at[idxd.at[j]], add=True)
            return carry

        lax.fori_loop(0, NCH, chunk, 0)
        plsc.subcore_barrier()

        pltpu.sync_copy(acc.at[pl.ds(w * RPT, RPT)],
                        out_hbm.at[pl.ds(c * Np + w * RPT, RPT)])

    return agg_kernel(table2n, srcp, dstp)


def _mlp_body(split_out, hp_ref, w1_ref, b1_ref, w2_ref, b2_ref, out_ref):
    hin = jnp.concatenate([hp_ref[0], hp_ref[1]], axis=1)
    h1 = jnp.maximum(
        jnp.dot(hin, w1_ref[...], preferred_element_type=jnp.float32)
        + b1_ref[...], 0.0)
    h2 = (jnp.dot(h1, w2_ref[...], preferred_element_type=jnp.float32)
          + b2_ref[...])
    if split_out:
        h2 = jnp.maximum(h2, 0.0)
        out_ref[0] = h2[:, :H]
        out_ref[1] = h2[:, H:]
    else:
        out_ref[...] = h2


def _tc_mlp(hp, w1, b1, w2, b2, split_out):
    if split_out:
        R = 640
        grid = (Np // R,)
        out_shape = jax.ShapeDtypeStruct((2, Np, H), jnp.float32)
        out_spec = pl.BlockSpec((2, R, H), lambda i: (0, i, 0))
    else:
        R = 1000
        grid = (N // R,)
        out_shape = jax.ShapeDtypeStruct((N, D), jnp.float32)
        out_spec = pl.BlockSpec((R, D), lambda i: (i, 0))
    return pl.pallas_call(
        functools.partial(_mlp_body, split_out),
        grid=grid,
        in_specs=[
            pl.BlockSpec((2, R, H), lambda i: (0, i, 0)),
            pl.BlockSpec((D, D), lambda i: (0, 0)),
            pl.BlockSpec((1, D), lambda i: (0, 0)),
            pl.BlockSpec((D, D), lambda i: (0, 0)),
            pl.BlockSpec((1, D), lambda i: (0, 0)),
        ],
        out_specs=out_spec,
        out_shape=out_shape,
    )(hp, w1, b1.reshape(1, D), w2, b2.reshape(1, D))


def kernel(x, edge_index, edge_weight, W1_0, b1_0, W2_0, b2_0,
           W1_1, b1_1, W2_1, b2_1, W1_2, b1_2, W2_2, b2_2):
    src = edge_index[0]
    dst = edge_index[1]
    srcp = jnp.stack([src, src + Np]).reshape(NC, NS, NCH, CH)
    dstp = dst.reshape(NS, NCH, CH)

    hs = jnp.zeros((2, Np, H), jnp.float32)
    hs = hs.at[0, :N].set(x[:, :H]).at[1, :N].set(x[:, H:]).reshape(2 * Np, H)
    params = [(W1_0, b1_0, W2_0, b2_0),
              (W1_1, b1_1, W2_1, b2_1),
              (W1_2, b1_2, W2_2, b2_2)]
    for l, (w1, b1, w2, b2) in enumerate(params):
        hp = _sc_aggregate(hs, srcp, dstp).reshape(2, Np, H)
        last = l == LAYERS - 1
        res = _tc_mlp(hp, w1, b1, w2, b2, split_out=not last)
        if last:
            return res
        hs = res.reshape(2 * Np, H)
